# baseline (speedup 1.0000x reference)
import jax
import jax.numpy as jnp
from jax import lax
from jax.experimental import pallas as pl
from jax.experimental.pallas import tpu as pltpu

VOCAB = 100000
EMBED_DIM = 64
CONTEXT = 50
HIDDEN = 128

BV = 8192
NB = (VOCAB + BV - 1) // BV

_NEG = -1e30


def _body(idx_ref, *refs):
    row_refs = refs[:CONTEXT]
    (w1_ref, b1_ref, w2_ref, b2_ref,
     o_ref, h_ref, m_ref, s_ref, lse_ref) = refs[CONTEXT:]
    phase = pl.program_id(0)
    j = pl.program_id(1)

    @pl.when((phase == 0) & (j == 0))
    def _():
        h = b1_ref[...]
        sub = lax.broadcasted_iota(jnp.int32, (8, 1), 0)
        for k in range(CONTEXT):
            slab = row_refs[k][...]
            row = jnp.sum(jnp.where(sub == idx_ref[k] % 8, slab, 0.0),
                          axis=0, keepdims=True)
            h = h + jnp.dot(row,
                            w1_ref[pl.ds(k * EMBED_DIM, EMBED_DIM), :],
                            preferred_element_type=jnp.float32)
        h_ref[...] = jnp.maximum(h, 0.0)
        m_ref[...] = jnp.full((1, BV), _NEG, jnp.float32)
        s_ref[...] = jnp.zeros((1, BV), jnp.float32)

    @pl.when(phase == 0)
    def _():
        logits = jnp.dot(h_ref[...], w2_ref[...],
                         preferred_element_type=jnp.float32) + b2_ref[...]
        col = j * BV + lax.broadcasted_iota(jnp.int32, (1, BV), 1)
        logits = jnp.where(col < VOCAB, logits, _NEG)
        o_ref[...] = logits

        m_old = m_ref[...]
        m_new = jnp.maximum(m_old, logits)
        s_ref[...] = (s_ref[...] * jnp.exp(m_old - m_new)
                      + jnp.exp(logits - m_new))
        m_ref[...] = m_new

        @pl.when(j == NB - 1)
        def _():
            m_vec = m_ref[...]
            mx = jnp.max(m_vec)
            tot = jnp.sum(s_ref[...] * jnp.exp(m_vec - mx))
            lse_ref[0] = mx + jnp.log(tot)

    @pl.when(phase == 1)
    def _():
        o_ref[...] = o_ref[...] - lse_ref[0]


def _row_spec(k):
    return pl.BlockSpec((8, EMBED_DIM),
                        lambda p, j, idx, _k=k: (idx[_k] // 8, 0))


def _w2_map(p, j, idx):
    return (0, jnp.where(p == 0, j, NB - 1))


_grid_spec = pltpu.PrefetchScalarGridSpec(
    num_scalar_prefetch=1,
    grid=(2, NB),
    in_specs=[
        *[_row_spec(k) for k in range(CONTEXT)],
        pl.BlockSpec((CONTEXT * EMBED_DIM, HIDDEN), lambda p, j, idx: (0, 0)),
        pl.BlockSpec((1, HIDDEN), lambda p, j, idx: (0, 0)),
        pl.BlockSpec((HIDDEN, BV), _w2_map),
        pl.BlockSpec((1, BV), _w2_map),
    ],
    out_specs=pl.BlockSpec((1, BV), lambda p, j, idx: (0, j)),
    scratch_shapes=[
        pltpu.VMEM((1, HIDDEN), jnp.float32),
        pltpu.VMEM((1, BV), jnp.float32),
        pltpu.VMEM((1, BV), jnp.float32),
        pltpu.SMEM((1,), jnp.float32),
    ],
)

_mlp_call = pl.pallas_call(
    _body,
    grid_spec=_grid_spec,
    out_shape=jax.ShapeDtypeStruct((1, NB * BV), jnp.float32),
)


def kernel(inputs, table, W1, b1, W2, b2):
    idx = inputs.astype(jnp.int32)
    out = _mlp_call(idx, *([table] * CONTEXT), W1, b1.reshape(1, HIDDEN),
                    W2, b2.reshape(1, VOCAB))
    return out[:, :VOCAB]


# speedup vs baseline: 1.1263x; 1.1263x over previous
import jax
import jax.numpy as jnp
from jax import lax
from jax.experimental import pallas as pl
from jax.experimental.pallas import tpu as pltpu

VOCAB = 100000
EMBED_DIM = 64
CONTEXT = 50
HIDDEN = 128

BV = 8192
NB = (VOCAB + BV - 1) // BV
VPAD = NB * BV

_NEG = -1e30


def _body(idx_ref, table_ref, w1_ref, b1_ref, w2_ref, b2_ref, o_ref,
          slabs_ref, h_ref, m_ref, s_ref, sem):
    j = pl.program_id(0)

    @pl.when(j == 0)
    def _():
        copies = []
        for k in range(CONTEXT):
            base = (idx_ref[k] // 8) * 8
            c = pltpu.make_async_copy(
                table_ref.at[pl.ds(base, 8), :],
                slabs_ref.at[pl.ds(8 * k, 8), :], sem)
            c.start()
            copies.append(c)
        for c in copies:
            c.wait()
        h = b1_ref[...]
        sub = lax.broadcasted_iota(jnp.int32, (8, 1), 0)
        for k in range(CONTEXT):
            slab = slabs_ref[pl.ds(8 * k, 8), :]
            row = jnp.sum(jnp.where(sub == idx_ref[k] % 8, slab, 0.0),
                          axis=0, keepdims=True)
            h = h + jnp.dot(row,
                            w1_ref[pl.ds(k * EMBED_DIM, EMBED_DIM), :],
                            preferred_element_type=jnp.float32)
        h_ref[...] = jnp.maximum(h, 0.0)
        m_ref[...] = jnp.full((1, BV), _NEG, jnp.float32)
        s_ref[...] = jnp.zeros((1, BV), jnp.float32)

    logits = jnp.dot(h_ref[...], w2_ref[...],
                     preferred_element_type=jnp.float32) + b2_ref[...]
    col = j * BV + lax.broadcasted_iota(jnp.int32, (1, BV), 1)
    logits = jnp.where(col < VOCAB, logits, _NEG)
    o_ref[:, pl.ds(j * BV, BV)] = logits

    m_old = m_ref[...]
    m_new = jnp.maximum(m_old, logits)
    s_ref[...] = s_ref[...] * jnp.exp(m_old - m_new) + jnp.exp(logits - m_new)
    m_ref[...] = m_new

    @pl.when(j == NB - 1)
    def _():
        m_vec = m_ref[...]
        mx = jnp.max(m_vec)
        tot = jnp.sum(s_ref[...] * jnp.exp(m_vec - mx))
        o_ref[...] = o_ref[...] - (mx + jnp.log(tot))


_grid_spec = pltpu.PrefetchScalarGridSpec(
    num_scalar_prefetch=1,
    grid=(NB,),
    in_specs=[
        pl.BlockSpec(memory_space=pltpu.HBM),
        pl.BlockSpec((CONTEXT * EMBED_DIM, HIDDEN), lambda j, idx: (0, 0)),
        pl.BlockSpec((1, HIDDEN), lambda j, idx: (0, 0)),
        pl.BlockSpec((HIDDEN, BV), lambda j, idx: (0, j)),
        pl.BlockSpec((1, BV), lambda j, idx: (0, j)),
    ],
    out_specs=pl.BlockSpec((1, VPAD), lambda j, idx: (0, 0)),
    scratch_shapes=[
        pltpu.VMEM((8 * CONTEXT, EMBED_DIM), jnp.float32),
        pltpu.VMEM((1, HIDDEN), jnp.float32),
        pltpu.VMEM((1, BV), jnp.float32),
        pltpu.VMEM((1, BV), jnp.float32),
        pltpu.SemaphoreType.DMA,
    ],
)

_mlp_call = pl.pallas_call(
    _body,
    grid_spec=_grid_spec,
    out_shape=jax.ShapeDtypeStruct((1, VPAD), jnp.float32),
)


def kernel(inputs, table, W1, b1, W2, b2):
    idx = inputs.astype(jnp.int32)
    out = _mlp_call(idx, table, W1, b1.reshape(1, HIDDEN),
                    W2, b2.reshape(1, VOCAB))
    return out[:, :VOCAB]


# speedup vs baseline: 1.1329x; 1.0058x over previous
import jax
import jax.numpy as jnp
from jax import lax
from jax.experimental import pallas as pl
from jax.experimental.pallas import tpu as pltpu

VOCAB = 100000
EMBED_DIM = 64
CONTEXT = 50
HIDDEN = 128

BV = 25088
NB = (VOCAB + BV - 1) // BV
VPAD = NB * BV

_NEG = -1e30


def _body(idx_ref, table_ref, w1_ref, b1_ref, w2_ref, b2_ref, o_ref,
          slabs_ref, h_ref, m_ref, s_ref, sem):
    j = pl.program_id(0)

    @pl.when(j == 0)
    def _():
        copies = []
        for k in range(CONTEXT):
            base = (idx_ref[k] // 8) * 8
            c = pltpu.make_async_copy(
                table_ref.at[pl.ds(base, 8), :],
                slabs_ref.at[pl.ds(8 * k, 8), :], sem)
            c.start()
            copies.append(c)
        for c in copies:
            c.wait()
        h = b1_ref[...]
        sub = lax.broadcasted_iota(jnp.int32, (8, 1), 0)
        for k in range(CONTEXT):
            slab = slabs_ref[pl.ds(8 * k, 8), :]
            row = jnp.sum(jnp.where(sub == idx_ref[k] % 8, slab, 0.0),
                          axis=0, keepdims=True)
            h = h + jnp.dot(row,
                            w1_ref[pl.ds(k * EMBED_DIM, EMBED_DIM), :],
                            preferred_element_type=jnp.float32)
        h_ref[...] = jnp.maximum(h, 0.0)
        m_ref[...] = jnp.full((1, BV), _NEG, jnp.float32)
        s_ref[...] = jnp.zeros((1, BV), jnp.float32)

    logits = jnp.dot(h_ref[...], w2_ref[...],
                     preferred_element_type=jnp.float32) + b2_ref[...]
    col = j * BV + lax.broadcasted_iota(jnp.int32, (1, BV), 1)
    logits = jnp.where(col < VOCAB, logits, _NEG)
    o_ref[:, pl.ds(j * BV, BV)] = logits

    m_old = m_ref[...]
    m_new = jnp.maximum(m_old, logits)
    s_ref[...] = s_ref[...] * jnp.exp(m_old - m_new) + jnp.exp(logits - m_new)
    m_ref[...] = m_new

    @pl.when(j == NB - 1)
    def _():
        m_vec = m_ref[...]
        mx = jnp.max(m_vec)
        tot = jnp.sum(s_ref[...] * jnp.exp(m_vec - mx))
        o_ref[...] = o_ref[...] - (mx + jnp.log(tot))


_grid_spec = pltpu.PrefetchScalarGridSpec(
    num_scalar_prefetch=1,
    grid=(NB,),
    in_specs=[
        pl.BlockSpec(memory_space=pltpu.HBM),
        pl.BlockSpec((CONTEXT * EMBED_DIM, HIDDEN), lambda j, idx: (0, 0)),
        pl.BlockSpec((1, HIDDEN), lambda j, idx: (0, 0)),
        pl.BlockSpec((HIDDEN, BV), lambda j, idx: (0, j)),
        pl.BlockSpec((1, BV), lambda j, idx: (0, j)),
    ],
    out_specs=pl.BlockSpec((1, VPAD), lambda j, idx: (0, 0)),
    scratch_shapes=[
        pltpu.VMEM((8 * CONTEXT, EMBED_DIM), jnp.float32),
        pltpu.VMEM((1, HIDDEN), jnp.float32),
        pltpu.VMEM((1, BV), jnp.float32),
        pltpu.VMEM((1, BV), jnp.float32),
        pltpu.SemaphoreType.DMA,
    ],
)

_mlp_call = pl.pallas_call(
    _body,
    grid_spec=_grid_spec,
    out_shape=jax.ShapeDtypeStruct((1, VPAD), jnp.float32),
)


def kernel(inputs, table, W1, b1, W2, b2):
    idx = inputs.astype(jnp.int32)
    out = _mlp_call(idx, table, W1, b1.reshape(1, HIDDEN),
                    W2, b2.reshape(1, VOCAB))
    return out[:, :VOCAB]
